# baseline (speedup 1.0000x reference)
import functools

import jax
import jax.numpy as jnp
from jax import lax
from jax.experimental import pallas as pl
from jax.experimental.pallas import tpu as pltpu
from jax.experimental.pallas import tpu_sc as plsc

N_NODES = 10000
N_PAD = 10240
E_EDGES = 320000
E_PAD = 327680
IN_DIM = 128
HID = 16
OUT_DIM = 3

CHUNK = 128
N_CORES = 2
N_SUB = 16
N_WORKERS = N_CORES * N_SUB
EDGES_PER_TILE = E_PAD // N_WORKERS
CHUNKS_PER_TILE = EDGES_PER_TILE // CHUNK
ROWS_PER_TILE = N_PAD // N_SUB

BLK = 512
GRID = N_PAD // BLK

_MESH = plsc.VectorSubcoreMesh(core_axis_name="c", subcore_axis_name="s")


NBUF = 8
GROUPS = CHUNKS_PER_TILE // NBUF


@functools.partial(
    pl.kernel,
    mesh=_MESH,
    out_type=jax.ShapeDtypeStruct((N_CORES, N_PAD, HID), jnp.float32),
    compiler_params=pltpu.CompilerParams(use_tc_tiling_on_sc=False),
    scratch_types=[
        pltpu.VMEM((CHUNKS_PER_TILE, CHUNK), jnp.int32),
        pltpu.VMEM((CHUNKS_PER_TILE, CHUNK), jnp.int32),
        pltpu.VMEM((2 * NBUF, CHUNK, HID), jnp.float32),
        pltpu.VMEM_SHARED((N_PAD, HID), jnp.float32),
        [pltpu.SemaphoreType.DMA] * (2 * NBUF),
    ],
)
def _msg_pass(table, src2d, dst2d, out, src_v, dst_v, rows_v, acc_sh, sems):
    cid = lax.axis_index("c")
    sid = lax.axis_index("s")
    wid = cid * N_SUB + sid

    pltpu.sync_copy(src2d.at[pl.ds(wid * CHUNKS_PER_TILE, CHUNKS_PER_TILE)],
                    src_v)
    pltpu.sync_copy(dst2d.at[pl.ds(wid * CHUNKS_PER_TILE, CHUNKS_PER_TILE)],
                    dst_v)

    def _zrow(i, carry):
        rows_v[0, i, :] = jnp.zeros((HID,), jnp.float32)
        return carry

    lax.fori_loop(0, CHUNK, _zrow, 0)
    for k in range(ROWS_PER_TILE // CHUNK):
        pltpu.sync_copy(
            rows_v.at[0],
            acc_sh.at[pl.ds(sid * ROWS_PER_TILE + k * CHUNK, CHUNK)])
    plsc.subcore_barrier()

    def _gather(c, slot):
        pltpu.make_async_copy(
            table.at[src_v.at[c]], rows_v.at[slot], sems[slot]).start()

    def _wait_gather(c, slot):
        pltpu.make_async_copy(
            table.at[src_v.at[c]], rows_v.at[slot], sems[slot]).wait()

    def _scatter(c, slot):
        pltpu.async_copy(
            rows_v.at[slot], acc_sh.at[dst_v.at[c]], sems[slot], add=True)

    def _wait_scatter(c, slot):
        pltpu.make_async_copy(
            rows_v.at[slot], acc_sh.at[dst_v.at[c]], sems[slot]).wait()

    for b in range(NBUF):
        _gather(b, b)

    def _phase(g, par):
        for b in range(NBUF):
            _wait_gather(g * NBUF + b, par * NBUF + b)
            _scatter(g * NBUF + b, par * NBUF + b)
        for b in range(NBUF):
            slot = (1 - par) * NBUF + b

            @pl.when(g >= 1)
            def _():
                _wait_scatter((g - 1) * NBUF + b, slot)

            @pl.when(g + 1 < GROUPS)
            def _():
                _gather((g + 1) * NBUF + b, slot)

    def _pair(t, carry):
        _phase(t * 2, 0)
        _phase(t * 2 + 1, 1)
        return carry

    lax.fori_loop(0, GROUPS // 2, _pair, 0)
    for b in range(NBUF):
        _wait_scatter((GROUPS - 1) * NBUF + b, NBUF + b)
    plsc.subcore_barrier()

    pltpu.sync_copy(
        acc_sh.at[pl.ds(sid * ROWS_PER_TILE, ROWS_PER_TILE)],
        out.at[cid, pl.ds(sid * ROWS_PER_TILE, ROWS_PER_TILE)])


@functools.partial(
    pl.kernel,
    mesh=_MESH,
    out_type=jax.ShapeDtypeStruct((N_CORES, N_PAD), jnp.float32),
    compiler_params=pltpu.CompilerParams(use_tc_tiling_on_sc=False),
    scratch_types=[
        pltpu.VMEM((CHUNKS_PER_TILE, CHUNK), jnp.int32),
        pltpu.VMEM((CHUNK,), jnp.float32),
        pltpu.VMEM_SHARED((N_PAD,), jnp.float32),
        [pltpu.SemaphoreType.DMA] * (2 * NBUF),
    ],
)
def _degrees(dst2d, out, dst_v, ones_v, deg_sh, sems):
    cid = lax.axis_index("c")
    sid = lax.axis_index("s")
    wid = cid * N_SUB + sid

    pltpu.sync_copy(dst2d.at[pl.ds(wid * CHUNKS_PER_TILE, CHUNKS_PER_TILE)],
                    dst_v)

    def _fill(val):
        def _f(i, carry):
            ones_v[pl.ds(i * 16, 16)] = jnp.full((16,), val, jnp.float32)
            return carry
        lax.fori_loop(0, CHUNK // 16, _f, 0)

    _fill(0.0)
    for k in range(ROWS_PER_TILE // CHUNK):
        pltpu.sync_copy(
            ones_v, deg_sh.at[pl.ds(sid * ROWS_PER_TILE + k * CHUNK, CHUNK)])
    _fill(1.0)
    plsc.subcore_barrier()

    def _phase(g, par):
        for b in range(NBUF):
            pltpu.async_copy(ones_v, deg_sh.at[dst_v.at[g * NBUF + b]],
                             sems[par * NBUF + b], add=True)
        for b in range(NBUF):
            @pl.when(g >= 1)
            def _():
                pltpu.make_async_copy(
                    ones_v, deg_sh.at[dst_v.at[(g - 1) * NBUF + b]],
                    sems[(1 - par) * NBUF + b]).wait()

    def _pair(t, carry):
        _phase(t * 2, 0)
        _phase(t * 2 + 1, 1)
        return carry

    lax.fori_loop(0, GROUPS // 2, _pair, 0)
    for b in range(NBUF):
        pltpu.make_async_copy(
            ones_v, deg_sh.at[dst_v.at[(GROUPS - 1) * NBUF + b]],
            sems[NBUF + b]).wait()
    plsc.subcore_barrier()

    pltpu.sync_copy(
        deg_sh.at[pl.ds(sid * ROWS_PER_TILE, ROWS_PER_TILE)],
        out.at[cid, pl.ds(sid * ROWS_PER_TILE, ROWS_PER_TILE)])


def _tc_a_body(deg_ref, x_ref, w1_ref, g1_ref, dinv_ref):
    deg = deg_ref[:, 0:1] + deg_ref[:, 1:2] + 1.0
    dinv = lax.rsqrt(deg)
    h = jnp.dot(x_ref[...], w1_ref[...], preferred_element_type=jnp.float32)
    g1_ref[...] = h * dinv
    dinv_ref[...] = dinv


def _tc_b_body(acc_ref, g1_ref, dinv_ref, b1_ref, g2_ref):
    s = acc_ref[0] + acc_ref[1] + g1_ref[...]
    pre = s * dinv_ref[...] + b1_ref[...]
    g2_ref[...] = jnp.maximum(pre, 0.0) * dinv_ref[...]


def _tc_c_body(acc_ref, g2_ref, dinv_ref, w2_ref, b2_ref, out_ref):
    s = (acc_ref[0] + acc_ref[1] + g2_ref[...]) * dinv_ref[...]
    out_ref[...] = (
        jnp.dot(s, w2_ref[...], preferred_element_type=jnp.float32)
        + b2_ref[...])


_tc_a = pl.pallas_call(
    _tc_a_body,
    grid=(GRID,),
    in_specs=[
        pl.BlockSpec((BLK, N_CORES), lambda i: (i, 0)),
        pl.BlockSpec((BLK, IN_DIM), lambda i: (i, 0)),
        pl.BlockSpec((IN_DIM, HID), lambda i: (0, 0)),
    ],
    out_specs=[
        pl.BlockSpec((BLK, HID), lambda i: (i, 0)),
        pl.BlockSpec((BLK, 1), lambda i: (i, 0)),
    ],
    out_shape=[
        jax.ShapeDtypeStruct((N_PAD, HID), jnp.float32),
        jax.ShapeDtypeStruct((N_PAD, 1), jnp.float32),
    ],
)

_tc_b = pl.pallas_call(
    _tc_b_body,
    grid=(GRID,),
    in_specs=[
        pl.BlockSpec((N_CORES, BLK, HID), lambda i: (0, i, 0)),
        pl.BlockSpec((BLK, HID), lambda i: (i, 0)),
        pl.BlockSpec((BLK, 1), lambda i: (i, 0)),
        pl.BlockSpec((1, HID), lambda i: (0, 0)),
    ],
    out_specs=pl.BlockSpec((BLK, HID), lambda i: (i, 0)),
    out_shape=jax.ShapeDtypeStruct((N_PAD, HID), jnp.float32),
)

_tc_c = pl.pallas_call(
    _tc_c_body,
    grid=(GRID,),
    in_specs=[
        pl.BlockSpec((N_CORES, BLK, HID), lambda i: (0, i, 0)),
        pl.BlockSpec((BLK, HID), lambda i: (i, 0)),
        pl.BlockSpec((BLK, 1), lambda i: (i, 0)),
        pl.BlockSpec((HID, OUT_DIM), lambda i: (0, 0)),
        pl.BlockSpec((1, OUT_DIM), lambda i: (0, 0)),
    ],
    out_specs=pl.BlockSpec((BLK, OUT_DIM), lambda i: (i, 0)),
    out_shape=jax.ShapeDtypeStruct((N_PAD, OUT_DIM), jnp.float32),
)


def kernel(x, edge_index, W1, b1, W2, b2):
    src = edge_index[0]
    dst = edge_index[1]
    pad = jnp.full((E_PAD - E_EDGES,), N_NODES, jnp.int32)
    src_p = jnp.concatenate([src, pad]).reshape(
        N_WORKERS * CHUNKS_PER_TILE, CHUNK)
    dst_p = jnp.concatenate([dst, pad]).reshape(
        N_WORKERS * CHUNKS_PER_TILE, CHUNK)
    x_p = jnp.pad(x, ((0, N_PAD - N_NODES), (0, 0)))

    deg_parts = _degrees(dst_p)
    g1, dinv = _tc_a(deg_parts.T, x_p, W1)
    acc1 = _msg_pass(g1, src_p, dst_p)
    g2 = _tc_b(acc1, g1, dinv, b1[None, :])
    acc2 = _msg_pass(g2, src_p, dst_p)
    out = _tc_c(acc2, g2, dinv, W2, b2[None, :])
    return out[:N_NODES]


# speedup vs baseline: 1.2458x; 1.2458x over previous
import functools

import jax
import jax.numpy as jnp
from jax import lax
from jax.experimental import pallas as pl
from jax.experimental.pallas import tpu as pltpu
from jax.experimental.pallas import tpu_sc as plsc

N_NODES = 10000
N_PAD = 10240
E_EDGES = 320000
E_PAD = 327680
IN_DIM = 128
HID = 16
OUT_DIM = 3

CHUNK = 128
N_CORES = 2
N_SUB = 16
N_WORKERS = N_CORES * N_SUB
EDGES_PER_TILE = E_PAD // N_WORKERS
CHUNKS_PER_TILE = EDGES_PER_TILE // CHUNK
ROWS_PER_TILE = N_PAD // N_SUB

BLK = 1000
GRID = N_NODES // BLK

_MESH = plsc.VectorSubcoreMesh(core_axis_name="c", subcore_axis_name="s")


NBUF = 8
GROUPS = CHUNKS_PER_TILE // NBUF

C0_CHUNKS = 128
C1_CHUNKS = 32
CMAX = max(C0_CHUNKS, C1_CHUNKS)
assert N_SUB * (C0_CHUNKS + C1_CHUNKS) == E_PAD // CHUNK
assert C0_CHUNKS % (2 * NBUF) == 0 and C1_CHUNKS % (2 * NBUF) == 0


def _chunk_layout(cid, sid):
    chunk0 = jnp.where(cid == 0, sid * C0_CHUNKS,
                       N_SUB * C0_CHUNKS + sid * C1_CHUNKS)
    n_groups = jnp.where(cid == 0, C0_CHUNKS // NBUF, C1_CHUNKS // NBUF)
    return chunk0, n_groups


@functools.partial(
    pl.kernel,
    mesh=_MESH,
    out_type=jax.ShapeDtypeStruct((N_CORES, N_PAD, HID), jnp.float32),
    compiler_params=pltpu.CompilerParams(use_tc_tiling_on_sc=False),
    scratch_types=[
        pltpu.VMEM((CMAX, CHUNK), jnp.int32),
        pltpu.VMEM((CMAX, CHUNK), jnp.int32),
        pltpu.VMEM((2 * NBUF, CHUNK, HID), jnp.float32),
        pltpu.VMEM_SHARED((N_PAD, HID), jnp.float32),
        [pltpu.SemaphoreType.DMA] * (2 * NBUF),
    ],
)
def _msg_pass(table, src2d, dst2d, out, src_v, dst_v, rows_v, acc_sh, sems):
    cid = lax.axis_index("c")
    sid = lax.axis_index("s")
    chunk0, n_groups = _chunk_layout(cid, sid)

    @pl.when(cid == 0)
    def _():
        pltpu.sync_copy(src2d.at[pl.ds(chunk0, C0_CHUNKS)],
                        src_v.at[pl.ds(0, C0_CHUNKS)])
        pltpu.sync_copy(dst2d.at[pl.ds(chunk0, C0_CHUNKS)],
                        dst_v.at[pl.ds(0, C0_CHUNKS)])

    @pl.when(cid == 1)
    def _():
        pltpu.sync_copy(src2d.at[pl.ds(chunk0, C1_CHUNKS)],
                        src_v.at[pl.ds(0, C1_CHUNKS)])
        pltpu.sync_copy(dst2d.at[pl.ds(chunk0, C1_CHUNKS)],
                        dst_v.at[pl.ds(0, C1_CHUNKS)])

    def _zrow(i, carry):
        rows_v[0, i, :] = jnp.zeros((HID,), jnp.float32)
        return carry

    lax.fori_loop(0, CHUNK, _zrow, 0)
    for k in range(ROWS_PER_TILE // CHUNK):
        pltpu.sync_copy(
            rows_v.at[0],
            acc_sh.at[pl.ds(sid * ROWS_PER_TILE + k * CHUNK, CHUNK)])
    plsc.subcore_barrier()

    def _gather(c, slot):
        pltpu.make_async_copy(
            table.at[src_v.at[c]], rows_v.at[slot], sems[slot]).start()

    def _wait_gather(c, slot):
        pltpu.make_async_copy(
            table.at[src_v.at[c]], rows_v.at[slot], sems[slot]).wait()

    def _scatter(c, slot):
        pltpu.async_copy(
            rows_v.at[slot], acc_sh.at[dst_v.at[c]], sems[slot], add=True)

    def _wait_scatter(c, slot):
        pltpu.make_async_copy(
            rows_v.at[slot], acc_sh.at[dst_v.at[c]], sems[slot]).wait()

    for b in range(NBUF):
        _gather(b, b)

    def _phase(g, par):
        for b in range(NBUF):
            _wait_gather(g * NBUF + b, par * NBUF + b)
            _scatter(g * NBUF + b, par * NBUF + b)
        for b in range(NBUF):
            slot = (1 - par) * NBUF + b

            @pl.when(g >= 1)
            def _():
                _wait_scatter((g - 1) * NBUF + b, slot)

            @pl.when(g + 1 < n_groups)
            def _():
                _gather((g + 1) * NBUF + b, slot)

    def _pair(t, carry):
        _phase(t * 2, 0)
        _phase(t * 2 + 1, 1)
        return carry

    lax.fori_loop(0, n_groups // 2, _pair, 0)
    for b in range(NBUF):
        _wait_scatter((n_groups - 1) * NBUF + b, NBUF + b)
    plsc.subcore_barrier()

    pltpu.sync_copy(
        acc_sh.at[pl.ds(sid * ROWS_PER_TILE, ROWS_PER_TILE)],
        out.at[cid, pl.ds(sid * ROWS_PER_TILE, ROWS_PER_TILE)])


@functools.partial(
    pl.kernel,
    mesh=_MESH,
    out_type=jax.ShapeDtypeStruct((N_CORES, N_PAD), jnp.float32),
    compiler_params=pltpu.CompilerParams(use_tc_tiling_on_sc=False),
    scratch_types=[
        pltpu.VMEM((CMAX, CHUNK), jnp.int32),
        pltpu.VMEM((CHUNK,), jnp.float32),
        pltpu.VMEM_SHARED((N_PAD,), jnp.float32),
        [pltpu.SemaphoreType.DMA] * (2 * NBUF),
    ],
)
def _degrees(dst2d, out, dst_v, ones_v, deg_sh, sems):
    cid = lax.axis_index("c")
    sid = lax.axis_index("s")
    chunk0, n_groups = _chunk_layout(cid, sid)

    @pl.when(cid == 0)
    def _():
        pltpu.sync_copy(dst2d.at[pl.ds(chunk0, C0_CHUNKS)],
                        dst_v.at[pl.ds(0, C0_CHUNKS)])

    @pl.when(cid == 1)
    def _():
        pltpu.sync_copy(dst2d.at[pl.ds(chunk0, C1_CHUNKS)],
                        dst_v.at[pl.ds(0, C1_CHUNKS)])

    def _fill(val):
        def _f(i, carry):
            ones_v[pl.ds(i * 16, 16)] = jnp.full((16,), val, jnp.float32)
            return carry
        lax.fori_loop(0, CHUNK // 16, _f, 0)

    _fill(0.0)
    for k in range(ROWS_PER_TILE // CHUNK):
        pltpu.sync_copy(
            ones_v, deg_sh.at[pl.ds(sid * ROWS_PER_TILE + k * CHUNK, CHUNK)])
    _fill(1.0)
    plsc.subcore_barrier()

    def _phase(g, par):
        for b in range(NBUF):
            pltpu.async_copy(ones_v, deg_sh.at[dst_v.at[g * NBUF + b]],
                             sems[par * NBUF + b], add=True)
        for b in range(NBUF):
            @pl.when(g >= 1)
            def _():
                pltpu.make_async_copy(
                    ones_v, deg_sh.at[dst_v.at[(g - 1) * NBUF + b]],
                    sems[(1 - par) * NBUF + b]).wait()

    def _pair(t, carry):
        _phase(t * 2, 0)
        _phase(t * 2 + 1, 1)
        return carry

    lax.fori_loop(0, n_groups // 2, _pair, 0)
    for b in range(NBUF):
        pltpu.make_async_copy(
            ones_v, deg_sh.at[dst_v.at[(n_groups - 1) * NBUF + b]],
            sems[NBUF + b]).wait()
    plsc.subcore_barrier()

    pltpu.sync_copy(
        deg_sh.at[pl.ds(sid * ROWS_PER_TILE, ROWS_PER_TILE)],
        out.at[cid, pl.ds(sid * ROWS_PER_TILE, ROWS_PER_TILE)])


def _tc_a_body(deg_ref, x_ref, w1_ref, g1_ref, dinv_ref):
    deg = deg_ref[:, 0:1] + deg_ref[:, 1:2] + 1.0
    dinv = lax.rsqrt(deg)
    h = jnp.dot(x_ref[...], w1_ref[...], preferred_element_type=jnp.float32)
    g1_ref[...] = h * dinv
    dinv_ref[...] = dinv


def _tc_b_body(acc_ref, g1_ref, dinv_ref, b1_ref, g2_ref):
    s = acc_ref[0] + acc_ref[1] + g1_ref[...]
    pre = s * dinv_ref[...] + b1_ref[...]
    g2_ref[...] = jnp.maximum(pre, 0.0) * dinv_ref[...]


def _tc_c_body(acc_ref, g2_ref, dinv_ref, w2_ref, b2_ref, out_ref):
    s = (acc_ref[0] + acc_ref[1] + g2_ref[...]) * dinv_ref[...]
    out_ref[...] = (
        jnp.dot(s, w2_ref[...], preferred_element_type=jnp.float32)
        + b2_ref[...])


_tc_a = pl.pallas_call(
    _tc_a_body,
    grid=(GRID,),
    in_specs=[
        pl.BlockSpec((BLK, N_CORES), lambda i: (i, 0)),
        pl.BlockSpec((BLK, IN_DIM), lambda i: (i, 0)),
        pl.BlockSpec((IN_DIM, HID), lambda i: (0, 0)),
    ],
    out_specs=[
        pl.BlockSpec((BLK, HID), lambda i: (i, 0)),
        pl.BlockSpec((BLK, 1), lambda i: (i, 0)),
    ],
    out_shape=[
        jax.ShapeDtypeStruct((N_NODES, HID), jnp.float32),
        jax.ShapeDtypeStruct((N_NODES, 1), jnp.float32),
    ],
)

_tc_b = pl.pallas_call(
    _tc_b_body,
    grid=(GRID,),
    in_specs=[
        pl.BlockSpec((N_CORES, BLK, HID), lambda i: (0, i, 0)),
        pl.BlockSpec((BLK, HID), lambda i: (i, 0)),
        pl.BlockSpec((BLK, 1), lambda i: (i, 0)),
        pl.BlockSpec((1, HID), lambda i: (0, 0)),
    ],
    out_specs=pl.BlockSpec((BLK, HID), lambda i: (i, 0)),
    out_shape=jax.ShapeDtypeStruct((N_NODES, HID), jnp.float32),
)

_tc_c = pl.pallas_call(
    _tc_c_body,
    grid=(GRID,),
    in_specs=[
        pl.BlockSpec((N_CORES, BLK, HID), lambda i: (0, i, 0)),
        pl.BlockSpec((BLK, HID), lambda i: (i, 0)),
        pl.BlockSpec((BLK, 1), lambda i: (i, 0)),
        pl.BlockSpec((HID, OUT_DIM), lambda i: (0, 0)),
        pl.BlockSpec((1, OUT_DIM), lambda i: (0, 0)),
    ],
    out_specs=pl.BlockSpec((BLK, OUT_DIM), lambda i: (i, 0)),
    out_shape=jax.ShapeDtypeStruct((N_NODES, OUT_DIM), jnp.float32),
)


def kernel(x, edge_index, W1, b1, W2, b2):
    src = edge_index[0]
    dst = edge_index[1]
    pad_src = jnp.zeros((E_PAD - E_EDGES,), jnp.int32)
    pad_dst = jnp.full((E_PAD - E_EDGES,), N_NODES, jnp.int32)
    src_p = jnp.concatenate([src, pad_src]).reshape(
        N_WORKERS * CHUNKS_PER_TILE, CHUNK)
    dst_p = jnp.concatenate([dst, pad_dst]).reshape(
        N_WORKERS * CHUNKS_PER_TILE, CHUNK)

    deg_parts = _degrees(dst_p)
    g1, dinv = _tc_a(deg_parts.T, x, W1)
    acc1 = _msg_pass(g1, src_p, dst_p)
    g2 = _tc_b(acc1, g1, dinv, b1[None, :])
    acc2 = _msg_pass(g2, src_p, dst_p)
    return _tc_c(acc2, g2, dinv, W2, b2[None, :])


# speedup vs baseline: 1.4878x; 1.1942x over previous
import functools

import jax
import jax.numpy as jnp
from jax import lax
from jax.experimental import pallas as pl
from jax.experimental.pallas import tpu as pltpu
from jax.experimental.pallas import tpu_sc as plsc

N_NODES = 10000
N_PAD = 10240
E_EDGES = 320000
E_PAD = 327680
IN_DIM = 128
HID = 16
OUT_DIM = 3

CHUNK = 128
N_CORES = 2
N_SUB = 16
N_WORKERS = N_CORES * N_SUB
EDGES_PER_TILE = E_PAD // N_WORKERS
CHUNKS_PER_TILE = EDGES_PER_TILE // CHUNK
ROWS_PER_TILE = N_PAD // N_SUB

BLK = 1000
GRID = N_NODES // BLK

_MESH = plsc.VectorSubcoreMesh(core_axis_name="c", subcore_axis_name="s")


NBUF = 8
GROUPS = CHUNKS_PER_TILE // NBUF

C0_CHUNKS = 128
C1_CHUNKS = 32
CMAX = max(C0_CHUNKS, C1_CHUNKS)
assert N_SUB * (C0_CHUNKS + C1_CHUNKS) == E_PAD // CHUNK
assert C0_CHUNKS % (2 * NBUF) == 0 and C1_CHUNKS % (2 * NBUF) == 0


def _chunk_layout(cid, sid):
    chunk0 = jnp.where(cid == 0, sid * C0_CHUNKS,
                       N_SUB * C0_CHUNKS + sid * C1_CHUNKS)
    n_groups = jnp.where(cid == 0, C0_CHUNKS // NBUF, C1_CHUNKS // NBUF)
    return chunk0, n_groups


@functools.partial(
    pl.kernel,
    mesh=_MESH,
    out_type=jax.ShapeDtypeStruct((N_CORES, N_PAD, HID), jnp.float32),
    compiler_params=pltpu.CompilerParams(use_tc_tiling_on_sc=False),
    scratch_types=[
        pltpu.VMEM((CMAX, CHUNK), jnp.int32),
        pltpu.VMEM((CMAX, CHUNK), jnp.int32),
        pltpu.VMEM((2 * NBUF, CHUNK, HID), jnp.float32),
        pltpu.VMEM_SHARED((N_PAD, HID), jnp.float32),
        pltpu.VMEM_SHARED((N_PAD, HID), jnp.float32),
        [pltpu.SemaphoreType.DMA] * (2 * NBUF),
    ],
)
def _msg_pass(table, src2d, dst2d, out, src_v, dst_v, rows_v, acc_sh,
              tab_sh, sems):
    cid = lax.axis_index("c")
    sid = lax.axis_index("s")
    chunk0, n_groups = _chunk_layout(cid, sid)

    pltpu.sync_copy(table.at[pl.ds(sid * (N_NODES // N_SUB), N_NODES // N_SUB)],
                    tab_sh.at[pl.ds(sid * (N_NODES // N_SUB), N_NODES // N_SUB)])

    @pl.when(cid == 0)
    def _():
        pltpu.sync_copy(src2d.at[pl.ds(chunk0, C0_CHUNKS)],
                        src_v.at[pl.ds(0, C0_CHUNKS)])
        pltpu.sync_copy(dst2d.at[pl.ds(chunk0, C0_CHUNKS)],
                        dst_v.at[pl.ds(0, C0_CHUNKS)])

    @pl.when(cid == 1)
    def _():
        pltpu.sync_copy(src2d.at[pl.ds(chunk0, C1_CHUNKS)],
                        src_v.at[pl.ds(0, C1_CHUNKS)])
        pltpu.sync_copy(dst2d.at[pl.ds(chunk0, C1_CHUNKS)],
                        dst_v.at[pl.ds(0, C1_CHUNKS)])

    def _zrow(i, carry):
        rows_v[0, i, :] = jnp.zeros((HID,), jnp.float32)
        return carry

    lax.fori_loop(0, CHUNK, _zrow, 0)
    for k in range(ROWS_PER_TILE // CHUNK):
        pltpu.sync_copy(
            rows_v.at[0],
            acc_sh.at[pl.ds(sid * ROWS_PER_TILE + k * CHUNK, CHUNK)])
    plsc.subcore_barrier()

    def _gather(c, slot):
        pltpu.make_async_copy(
            tab_sh.at[src_v.at[c]], rows_v.at[slot], sems[slot]).start()

    def _wait_gather(c, slot):
        pltpu.make_async_copy(
            tab_sh.at[src_v.at[c]], rows_v.at[slot], sems[slot]).wait()

    def _scatter(c, slot):
        pltpu.async_copy(
            rows_v.at[slot], acc_sh.at[dst_v.at[c]], sems[slot], add=True)

    def _wait_scatter(c, slot):
        pltpu.make_async_copy(
            rows_v.at[slot], acc_sh.at[dst_v.at[c]], sems[slot]).wait()

    for b in range(NBUF):
        _gather(b, b)

    def _phase(g, par):
        for b in range(NBUF):
            _wait_gather(g * NBUF + b, par * NBUF + b)
            _scatter(g * NBUF + b, par * NBUF + b)
        for b in range(NBUF):
            slot = (1 - par) * NBUF + b

            @pl.when(g >= 1)
            def _():
                _wait_scatter((g - 1) * NBUF + b, slot)

            @pl.when(g + 1 < n_groups)
            def _():
                _gather((g + 1) * NBUF + b, slot)

    def _pair(t, carry):
        _phase(t * 2, 0)
        _phase(t * 2 + 1, 1)
        return carry

    lax.fori_loop(0, n_groups // 2, _pair, 0)
    for b in range(NBUF):
        _wait_scatter((n_groups - 1) * NBUF + b, NBUF + b)
    plsc.subcore_barrier()

    pltpu.sync_copy(
        acc_sh.at[pl.ds(sid * ROWS_PER_TILE, ROWS_PER_TILE)],
        out.at[cid, pl.ds(sid * ROWS_PER_TILE, ROWS_PER_TILE)])


@functools.partial(
    pl.kernel,
    mesh=_MESH,
    out_type=jax.ShapeDtypeStruct((N_CORES, N_PAD), jnp.float32),
    compiler_params=pltpu.CompilerParams(use_tc_tiling_on_sc=False),
    scratch_types=[
        pltpu.VMEM((CMAX, CHUNK), jnp.int32),
        pltpu.VMEM((CHUNK,), jnp.float32),
        pltpu.VMEM_SHARED((N_PAD,), jnp.float32),
        [pltpu.SemaphoreType.DMA] * (2 * NBUF),
    ],
)
def _degrees(dst2d, out, dst_v, ones_v, deg_sh, sems):
    cid = lax.axis_index("c")
    sid = lax.axis_index("s")
    chunk0, n_groups = _chunk_layout(cid, sid)

    @pl.when(cid == 0)
    def _():
        pltpu.sync_copy(dst2d.at[pl.ds(chunk0, C0_CHUNKS)],
                        dst_v.at[pl.ds(0, C0_CHUNKS)])

    @pl.when(cid == 1)
    def _():
        pltpu.sync_copy(dst2d.at[pl.ds(chunk0, C1_CHUNKS)],
                        dst_v.at[pl.ds(0, C1_CHUNKS)])

    def _fill(val):
        def _f(i, carry):
            ones_v[pl.ds(i * 16, 16)] = jnp.full((16,), val, jnp.float32)
            return carry
        lax.fori_loop(0, CHUNK // 16, _f, 0)

    _fill(0.0)
    for k in range(ROWS_PER_TILE // CHUNK):
        pltpu.sync_copy(
            ones_v, deg_sh.at[pl.ds(sid * ROWS_PER_TILE + k * CHUNK, CHUNK)])
    _fill(1.0)
    plsc.subcore_barrier()

    def _phase(g, par):
        for b in range(NBUF):
            pltpu.async_copy(ones_v, deg_sh.at[dst_v.at[g * NBUF + b]],
                             sems[par * NBUF + b], add=True)
        for b in range(NBUF):
            @pl.when(g >= 1)
            def _():
                pltpu.make_async_copy(
                    ones_v, deg_sh.at[dst_v.at[(g - 1) * NBUF + b]],
                    sems[(1 - par) * NBUF + b]).wait()

    def _pair(t, carry):
        _phase(t * 2, 0)
        _phase(t * 2 + 1, 1)
        return carry

    lax.fori_loop(0, n_groups // 2, _pair, 0)
    for b in range(NBUF):
        pltpu.make_async_copy(
            ones_v, deg_sh.at[dst_v.at[(n_groups - 1) * NBUF + b]],
            sems[NBUF + b]).wait()
    plsc.subcore_barrier()

    pltpu.sync_copy(
        deg_sh.at[pl.ds(sid * ROWS_PER_TILE, ROWS_PER_TILE)],
        out.at[cid, pl.ds(sid * ROWS_PER_TILE, ROWS_PER_TILE)])


def _tc_a_body(deg_ref, x_ref, w1_ref, g1_ref, dinv_ref):
    deg = deg_ref[:, 0:1] + deg_ref[:, 1:2] + 1.0
    dinv = lax.rsqrt(deg)
    h = jnp.dot(x_ref[...], w1_ref[...], preferred_element_type=jnp.float32)
    g1_ref[...] = h * dinv
    dinv_ref[...] = dinv


def _tc_b_body(acc_ref, g1_ref, dinv_ref, b1_ref, g2_ref):
    s = acc_ref[0] + acc_ref[1] + g1_ref[...]
    pre = s * dinv_ref[...] + b1_ref[...]
    g2_ref[...] = jnp.maximum(pre, 0.0) * dinv_ref[...]


def _tc_c_body(acc_ref, g2_ref, dinv_ref, w2_ref, b2_ref, out_ref):
    s = (acc_ref[0] + acc_ref[1] + g2_ref[...]) * dinv_ref[...]
    out_ref[...] = (
        jnp.dot(s, w2_ref[...], preferred_element_type=jnp.float32)
        + b2_ref[...])


_tc_a = pl.pallas_call(
    _tc_a_body,
    grid=(GRID,),
    in_specs=[
        pl.BlockSpec((BLK, N_CORES), lambda i: (i, 0)),
        pl.BlockSpec((BLK, IN_DIM), lambda i: (i, 0)),
        pl.BlockSpec((IN_DIM, HID), lambda i: (0, 0)),
    ],
    out_specs=[
        pl.BlockSpec((BLK, HID), lambda i: (i, 0)),
        pl.BlockSpec((BLK, 1), lambda i: (i, 0)),
    ],
    out_shape=[
        jax.ShapeDtypeStruct((N_NODES, HID), jnp.float32),
        jax.ShapeDtypeStruct((N_NODES, 1), jnp.float32),
    ],
)

_tc_b = pl.pallas_call(
    _tc_b_body,
    grid=(GRID,),
    in_specs=[
        pl.BlockSpec((N_CORES, BLK, HID), lambda i: (0, i, 0)),
        pl.BlockSpec((BLK, HID), lambda i: (i, 0)),
        pl.BlockSpec((BLK, 1), lambda i: (i, 0)),
        pl.BlockSpec((1, HID), lambda i: (0, 0)),
    ],
    out_specs=pl.BlockSpec((BLK, HID), lambda i: (i, 0)),
    out_shape=jax.ShapeDtypeStruct((N_NODES, HID), jnp.float32),
)

_tc_c = pl.pallas_call(
    _tc_c_body,
    grid=(GRID,),
    in_specs=[
        pl.BlockSpec((N_CORES, BLK, HID), lambda i: (0, i, 0)),
        pl.BlockSpec((BLK, HID), lambda i: (i, 0)),
        pl.BlockSpec((BLK, 1), lambda i: (i, 0)),
        pl.BlockSpec((HID, OUT_DIM), lambda i: (0, 0)),
        pl.BlockSpec((1, OUT_DIM), lambda i: (0, 0)),
    ],
    out_specs=pl.BlockSpec((BLK, OUT_DIM), lambda i: (i, 0)),
    out_shape=jax.ShapeDtypeStruct((N_NODES, OUT_DIM), jnp.float32),
)


def kernel(x, edge_index, W1, b1, W2, b2):
    src = edge_index[0]
    dst = edge_index[1]
    pad_src = jnp.zeros((E_PAD - E_EDGES,), jnp.int32)
    pad_dst = jnp.full((E_PAD - E_EDGES,), N_NODES, jnp.int32)
    src_p = jnp.concatenate([src, pad_src]).reshape(
        N_WORKERS * CHUNKS_PER_TILE, CHUNK)
    dst_p = jnp.concatenate([dst, pad_dst]).reshape(
        N_WORKERS * CHUNKS_PER_TILE, CHUNK)

    deg_parts = _degrees(dst_p)
    g1, dinv = _tc_a(deg_parts.T, x, W1)
    acc1 = _msg_pass(g1, src_p, dst_p)
    g2 = _tc_b(acc1, g1, dinv, b1[None, :])
    acc2 = _msg_pass(g2, src_p, dst_p)
    return _tc_c(acc2, g2, dinv, W2, b2[None, :])


# speedup vs baseline: 1.5491x; 1.0412x over previous
import functools

import jax
import jax.numpy as jnp
from jax import lax
from jax.experimental import pallas as pl
from jax.experimental.pallas import tpu as pltpu
from jax.experimental.pallas import tpu_sc as plsc

N_NODES = 10000
N_PAD = 10240
E_EDGES = 320000
E_PAD = 327680
IN_DIM = 128
HID = 16
OUT_DIM = 3

CHUNK = 128
N_CORES = 2
N_SUB = 16
N_WORKERS = N_CORES * N_SUB
EDGES_PER_TILE = E_PAD // N_WORKERS
CHUNKS_PER_TILE = EDGES_PER_TILE // CHUNK
ROWS_PER_TILE = N_PAD // N_SUB

BLK = 1000
GRID = N_NODES // BLK

_MESH = plsc.VectorSubcoreMesh(core_axis_name="c", subcore_axis_name="s")


NBUF = 8
GROUPS = CHUNKS_PER_TILE // NBUF

C0_CHUNKS = 112
C1_CHUNKS = 48
CMAX = max(C0_CHUNKS, C1_CHUNKS)
assert N_SUB * (C0_CHUNKS + C1_CHUNKS) == E_PAD // CHUNK
assert C0_CHUNKS % (2 * NBUF) == 0 and C1_CHUNKS % (2 * NBUF) == 0


def _chunk_layout(cid, sid):
    chunk0 = jnp.where(cid == 0, sid * C0_CHUNKS,
                       N_SUB * C0_CHUNKS + sid * C1_CHUNKS)
    n_groups = jnp.where(cid == 0, C0_CHUNKS // NBUF, C1_CHUNKS // NBUF)
    return chunk0, n_groups


@functools.partial(
    pl.kernel,
    mesh=_MESH,
    out_type=jax.ShapeDtypeStruct((N_CORES, N_PAD, HID), jnp.float32),
    compiler_params=pltpu.CompilerParams(use_tc_tiling_on_sc=False),
    scratch_types=[
        pltpu.VMEM((CMAX, CHUNK), jnp.int32),
        pltpu.VMEM((CMAX, CHUNK), jnp.int32),
        pltpu.VMEM((2 * NBUF, CHUNK, HID), jnp.float32),
        pltpu.VMEM_SHARED((N_PAD, HID), jnp.float32),
        pltpu.VMEM_SHARED((N_PAD, HID), jnp.float32),
        [pltpu.SemaphoreType.DMA] * (2 * NBUF),
    ],
)
def _msg_pass(table, src2d, dst2d, out, src_v, dst_v, rows_v, acc_sh,
              tab_sh, sems):
    cid = lax.axis_index("c")
    sid = lax.axis_index("s")
    chunk0, n_groups = _chunk_layout(cid, sid)

    pltpu.sync_copy(table.at[pl.ds(sid * (N_NODES // N_SUB), N_NODES // N_SUB)],
                    tab_sh.at[pl.ds(sid * (N_NODES // N_SUB), N_NODES // N_SUB)])

    @pl.when(cid == 0)
    def _():
        pltpu.sync_copy(src2d.at[pl.ds(chunk0, C0_CHUNKS)],
                        src_v.at[pl.ds(0, C0_CHUNKS)])
        pltpu.sync_copy(dst2d.at[pl.ds(chunk0, C0_CHUNKS)],
                        dst_v.at[pl.ds(0, C0_CHUNKS)])

    @pl.when(cid == 1)
    def _():
        pltpu.sync_copy(src2d.at[pl.ds(chunk0, C1_CHUNKS)],
                        src_v.at[pl.ds(0, C1_CHUNKS)])
        pltpu.sync_copy(dst2d.at[pl.ds(chunk0, C1_CHUNKS)],
                        dst_v.at[pl.ds(0, C1_CHUNKS)])

    def _zrow(i, carry):
        rows_v[0, i, :] = jnp.zeros((HID,), jnp.float32)
        return carry

    lax.fori_loop(0, CHUNK, _zrow, 0)
    for k in range(ROWS_PER_TILE // CHUNK):
        pltpu.sync_copy(
            rows_v.at[0],
            acc_sh.at[pl.ds(sid * ROWS_PER_TILE + k * CHUNK, CHUNK)])
    plsc.subcore_barrier()

    def _gather(c, slot):
        pltpu.make_async_copy(
            tab_sh.at[src_v.at[c]], rows_v.at[slot], sems[slot]).start()

    def _wait_gather(c, slot):
        pltpu.make_async_copy(
            tab_sh.at[src_v.at[c]], rows_v.at[slot], sems[slot]).wait()

    def _scatter(c, slot):
        pltpu.async_copy(
            rows_v.at[slot], acc_sh.at[dst_v.at[c]], sems[slot], add=True)

    def _wait_scatter(c, slot):
        pltpu.make_async_copy(
            rows_v.at[slot], acc_sh.at[dst_v.at[c]], sems[slot]).wait()

    for b in range(NBUF):
        _gather(b, b)

    def _phase(g, par):
        for b in range(NBUF):
            _wait_gather(g * NBUF + b, par * NBUF + b)
            _scatter(g * NBUF + b, par * NBUF + b)
        for b in range(NBUF):
            slot = (1 - par) * NBUF + b

            @pl.when(g >= 1)
            def _():
                _wait_scatter((g - 1) * NBUF + b, slot)

            @pl.when(g + 1 < n_groups)
            def _():
                _gather((g + 1) * NBUF + b, slot)

    def _pair(t, carry):
        _phase(t * 2, 0)
        _phase(t * 2 + 1, 1)
        return carry

    lax.fori_loop(0, n_groups // 2, _pair, 0)
    for b in range(NBUF):
        _wait_scatter((n_groups - 1) * NBUF + b, NBUF + b)
    plsc.subcore_barrier()

    pltpu.sync_copy(
        acc_sh.at[pl.ds(sid * ROWS_PER_TILE, ROWS_PER_TILE)],
        out.at[cid, pl.ds(sid * ROWS_PER_TILE, ROWS_PER_TILE)])


@functools.partial(
    pl.kernel,
    mesh=_MESH,
    out_type=jax.ShapeDtypeStruct((N_CORES, N_PAD), jnp.float32),
    compiler_params=pltpu.CompilerParams(use_tc_tiling_on_sc=False),
    scratch_types=[
        pltpu.VMEM((CMAX, CHUNK), jnp.int32),
        pltpu.VMEM((CHUNK,), jnp.float32),
        pltpu.VMEM_SHARED((N_PAD,), jnp.float32),
        [pltpu.SemaphoreType.DMA] * (2 * NBUF),
    ],
)
def _degrees(dst2d, out, dst_v, ones_v, deg_sh, sems):
    cid = lax.axis_index("c")
    sid = lax.axis_index("s")
    chunk0, n_groups = _chunk_layout(cid, sid)

    @pl.when(cid == 0)
    def _():
        pltpu.sync_copy(dst2d.at[pl.ds(chunk0, C0_CHUNKS)],
                        dst_v.at[pl.ds(0, C0_CHUNKS)])

    @pl.when(cid == 1)
    def _():
        pltpu.sync_copy(dst2d.at[pl.ds(chunk0, C1_CHUNKS)],
                        dst_v.at[pl.ds(0, C1_CHUNKS)])

    def _fill(val):
        def _f(i, carry):
            ones_v[pl.ds(i * 16, 16)] = jnp.full((16,), val, jnp.float32)
            return carry
        lax.fori_loop(0, CHUNK // 16, _f, 0)

    _fill(0.0)
    for k in range(ROWS_PER_TILE // CHUNK):
        pltpu.sync_copy(
            ones_v, deg_sh.at[pl.ds(sid * ROWS_PER_TILE + k * CHUNK, CHUNK)])
    _fill(1.0)
    plsc.subcore_barrier()

    def _phase(g, par):
        for b in range(NBUF):
            pltpu.async_copy(ones_v, deg_sh.at[dst_v.at[g * NBUF + b]],
                             sems[par * NBUF + b], add=True)
        for b in range(NBUF):
            @pl.when(g >= 1)
            def _():
                pltpu.make_async_copy(
                    ones_v, deg_sh.at[dst_v.at[(g - 1) * NBUF + b]],
                    sems[(1 - par) * NBUF + b]).wait()

    def _pair(t, carry):
        _phase(t * 2, 0)
        _phase(t * 2 + 1, 1)
        return carry

    lax.fori_loop(0, n_groups // 2, _pair, 0)
    for b in range(NBUF):
        pltpu.make_async_copy(
            ones_v, deg_sh.at[dst_v.at[(n_groups - 1) * NBUF + b]],
            sems[NBUF + b]).wait()
    plsc.subcore_barrier()

    pltpu.sync_copy(
        deg_sh.at[pl.ds(sid * ROWS_PER_TILE, ROWS_PER_TILE)],
        out.at[cid, pl.ds(sid * ROWS_PER_TILE, ROWS_PER_TILE)])


def _tc_a_body(deg_ref, x_ref, w1_ref, g1_ref, dinv_ref):
    deg = deg_ref[:, 0:1] + deg_ref[:, 1:2] + 1.0
    dinv = lax.rsqrt(deg)
    h = jnp.dot(x_ref[...], w1_ref[...], preferred_element_type=jnp.float32)
    g1_ref[...] = h * dinv
    dinv_ref[...] = dinv


def _tc_b_body(acc_ref, g1_ref, dinv_ref, b1_ref, g2_ref):
    s = acc_ref[0] + acc_ref[1] + g1_ref[...]
    pre = s * dinv_ref[...] + b1_ref[...]
    g2_ref[...] = jnp.maximum(pre, 0.0) * dinv_ref[...]


def _tc_c_body(acc_ref, g2_ref, dinv_ref, w2_ref, b2_ref, out_ref):
    s = (acc_ref[0] + acc_ref[1] + g2_ref[...]) * dinv_ref[...]
    out_ref[...] = (
        jnp.dot(s, w2_ref[...], preferred_element_type=jnp.float32)
        + b2_ref[...])


_tc_a = pl.pallas_call(
    _tc_a_body,
    grid=(GRID,),
    in_specs=[
        pl.BlockSpec((BLK, N_CORES), lambda i: (i, 0)),
        pl.BlockSpec((BLK, IN_DIM), lambda i: (i, 0)),
        pl.BlockSpec((IN_DIM, HID), lambda i: (0, 0)),
    ],
    out_specs=[
        pl.BlockSpec((BLK, HID), lambda i: (i, 0)),
        pl.BlockSpec((BLK, 1), lambda i: (i, 0)),
    ],
    out_shape=[
        jax.ShapeDtypeStruct((N_NODES, HID), jnp.float32),
        jax.ShapeDtypeStruct((N_NODES, 1), jnp.float32),
    ],
)

_tc_b = pl.pallas_call(
    _tc_b_body,
    grid=(GRID,),
    in_specs=[
        pl.BlockSpec((N_CORES, BLK, HID), lambda i: (0, i, 0)),
        pl.BlockSpec((BLK, HID), lambda i: (i, 0)),
        pl.BlockSpec((BLK, 1), lambda i: (i, 0)),
        pl.BlockSpec((1, HID), lambda i: (0, 0)),
    ],
    out_specs=pl.BlockSpec((BLK, HID), lambda i: (i, 0)),
    out_shape=jax.ShapeDtypeStruct((N_NODES, HID), jnp.float32),
)

_tc_c = pl.pallas_call(
    _tc_c_body,
    grid=(GRID,),
    in_specs=[
        pl.BlockSpec((N_CORES, BLK, HID), lambda i: (0, i, 0)),
        pl.BlockSpec((BLK, HID), lambda i: (i, 0)),
        pl.BlockSpec((BLK, 1), lambda i: (i, 0)),
        pl.BlockSpec((HID, OUT_DIM), lambda i: (0, 0)),
        pl.BlockSpec((1, OUT_DIM), lambda i: (0, 0)),
    ],
    out_specs=pl.BlockSpec((BLK, OUT_DIM), lambda i: (i, 0)),
    out_shape=jax.ShapeDtypeStruct((N_NODES, OUT_DIM), jnp.float32),
)


def kernel(x, edge_index, W1, b1, W2, b2):
    src = edge_index[0]
    dst = edge_index[1]
    pad_src = jnp.zeros((E_PAD - E_EDGES,), jnp.int32)
    pad_dst = jnp.full((E_PAD - E_EDGES,), N_NODES, jnp.int32)
    src_p = jnp.concatenate([src, pad_src]).reshape(
        N_WORKERS * CHUNKS_PER_TILE, CHUNK)
    dst_p = jnp.concatenate([dst, pad_dst]).reshape(
        N_WORKERS * CHUNKS_PER_TILE, CHUNK)

    deg_parts = _degrees(dst_p)
    g1, dinv = _tc_a(deg_parts.T, x, W1)
    acc1 = _msg_pass(g1, src_p, dst_p)
    g2 = _tc_b(acc1, g1, dinv, b1[None, :])
    acc2 = _msg_pass(g2, src_p, dst_p)
    return _tc_c(acc2, g2, dinv, W2, b2[None, :])


# speedup vs baseline: 1.6205x; 1.0461x over previous
import functools

import jax
import jax.numpy as jnp
from jax import lax
from jax.experimental import pallas as pl
from jax.experimental.pallas import tpu as pltpu
from jax.experimental.pallas import tpu_sc as plsc

N_NODES = 10000
N_PAD = 10240
E_EDGES = 320000
E_PAD = 327680
IN_DIM = 128
HID = 16
OUT_DIM = 3

CHUNK = 128
N_CORES = 2
N_SUB = 16
ROWS_PER_TILE = N_PAD // N_SUB
STAGE_ROWS = N_NODES // N_SUB
SCHUNK = 125

BLK = 1000
GRID = N_NODES // BLK

_MESH = plsc.VectorSubcoreMesh(core_axis_name="c", subcore_axis_name="s")


NBUF = 8

C0_CHUNKS = 112
C1_CHUNKS = 48
CMAX = max(C0_CHUNKS, C1_CHUNKS)
assert N_SUB * (C0_CHUNKS + C1_CHUNKS) == E_PAD // CHUNK
assert C0_CHUNKS % (2 * NBUF) == 0 and C1_CHUNKS % (2 * NBUF) == 0


def _chunk_layout(cid, sid):
    chunk0 = jnp.where(cid == 0, sid * C0_CHUNKS,
                       N_SUB * C0_CHUNKS + sid * C1_CHUNKS)
    n_groups = jnp.where(cid == 0, C0_CHUNKS // NBUF, C1_CHUNKS // NBUF)
    return chunk0, n_groups


def _load_idx_chunks(cid, chunk0, src2d, dst2d, src_v, dst_v):
    @pl.when(cid == 0)
    def _():
        pltpu.sync_copy(src2d.at[pl.ds(chunk0, C0_CHUNKS)],
                        src_v.at[pl.ds(0, C0_CHUNKS)])
        pltpu.sync_copy(dst2d.at[pl.ds(chunk0, C0_CHUNKS)],
                        dst_v.at[pl.ds(0, C0_CHUNKS)])

    @pl.when(cid == 1)
    def _():
        pltpu.sync_copy(src2d.at[pl.ds(chunk0, C1_CHUNKS)],
                        src_v.at[pl.ds(0, C1_CHUNKS)])
        pltpu.sync_copy(dst2d.at[pl.ds(chunk0, C1_CHUNKS)],
                        dst_v.at[pl.ds(0, C1_CHUNKS)])


def _zero_acc(cid, sid, rows_v, acc_sh):
    def _zrow(i, carry):
        rows_v[0, i, :] = jnp.zeros((HID,), jnp.float32)
        return carry

    lax.fori_loop(0, CHUNK, _zrow, 0)

    @pl.when(cid == 1)
    def _():
        for k in range(ROWS_PER_TILE // CHUNK):
            pltpu.sync_copy(
                rows_v.at[0],
                acc_sh.at[pl.ds(sid * ROWS_PER_TILE + k * CHUNK, CHUNK)])

    @pl.when((cid == 0) & (sid == N_SUB - 1))
    def _():
        pltpu.sync_copy(rows_v.at[0], acc_sh.at[pl.ds(N_NODES, CHUNK)])
        pltpu.sync_copy(
            rows_v.at[0, pl.ds(0, N_PAD - N_NODES - CHUNK)],
            acc_sh.at[pl.ds(N_NODES + CHUNK, N_PAD - N_NODES - CHUNK)])


def _msg_loop(cid, sid, out, src_v, dst_v, rows_v, acc_sh, tab_sh, sems,
              n_groups):
    def _gather(c, slot):
        pltpu.make_async_copy(
            tab_sh.at[src_v.at[c]], rows_v.at[slot], sems[slot]).start()

    def _wait_gather(c, slot):
        pltpu.make_async_copy(
            tab_sh.at[src_v.at[c]], rows_v.at[slot], sems[slot]).wait()

    def _scatter(c, slot):
        pltpu.async_copy(
            rows_v.at[slot], acc_sh.at[dst_v.at[c]], sems[slot], add=True)

    def _wait_scatter(c, slot):
        pltpu.make_async_copy(
            rows_v.at[slot], acc_sh.at[dst_v.at[c]], sems[slot]).wait()

    for b in range(NBUF):
        _gather(b, b)

    def _phase(g, par):
        for b in range(NBUF):
            _wait_gather(g * NBUF + b, par * NBUF + b)
            _scatter(g * NBUF + b, par * NBUF + b)
        for b in range(NBUF):
            slot = (1 - par) * NBUF + b

            @pl.when(g >= 1)
            def _():
                _wait_scatter((g - 1) * NBUF + b, slot)

            @pl.when(g + 1 < n_groups)
            def _():
                _gather((g + 1) * NBUF + b, slot)

    def _pair(t, carry):
        _phase(t * 2, 0)
        _phase(t * 2 + 1, 1)
        return carry

    lax.fori_loop(0, n_groups // 2, _pair, 0)
    for b in range(NBUF):
        _wait_scatter((n_groups - 1) * NBUF + b, NBUF + b)
    plsc.subcore_barrier()

    pltpu.sync_copy(
        acc_sh.at[pl.ds(sid * ROWS_PER_TILE, ROWS_PER_TILE)],
        out.at[cid, pl.ds(sid * ROWS_PER_TILE, ROWS_PER_TILE)])


@functools.partial(
    pl.kernel,
    mesh=_MESH,
    out_type=jax.ShapeDtypeStruct((N_CORES, N_PAD, HID), jnp.float32),
    compiler_params=pltpu.CompilerParams(use_tc_tiling_on_sc=False),
    scratch_types=[
        pltpu.VMEM((CMAX, CHUNK), jnp.int32),
        pltpu.VMEM((CMAX, CHUNK), jnp.int32),
        pltpu.VMEM((2 * NBUF, CHUNK, HID), jnp.float32),
        pltpu.VMEM_SHARED((N_PAD, HID), jnp.float32),
        pltpu.VMEM_SHARED((N_PAD, HID), jnp.float32),
        [pltpu.SemaphoreType.DMA] * (2 * NBUF),
    ],
)
def _msg_pass1(table, src2d, dst2d, out, src_v, dst_v, rows_v, acc_sh,
               tab_sh, sems):
    cid = lax.axis_index("c")
    sid = lax.axis_index("s")
    chunk0, n_groups = _chunk_layout(cid, sid)
    _load_idx_chunks(cid, chunk0, src2d, dst2d, src_v, dst_v)

    r0 = sid * STAGE_ROWS
    pltpu.sync_copy(table.at[pl.ds(r0, STAGE_ROWS)],
                    tab_sh.at[pl.ds(r0, STAGE_ROWS)])

    @pl.when(cid == 0)
    def _():
        pltpu.sync_copy(table.at[pl.ds(r0, STAGE_ROWS)],
                        acc_sh.at[pl.ds(r0, STAGE_ROWS)])

    _zero_acc(cid, sid, rows_v, acc_sh)
    plsc.subcore_barrier()
    _msg_loop(cid, sid, out, src_v, dst_v, rows_v, acc_sh, tab_sh, sems,
              n_groups)


@functools.partial(
    pl.kernel,
    mesh=_MESH,
    out_type=jax.ShapeDtypeStruct((N_CORES, N_PAD, HID), jnp.float32),
    compiler_params=pltpu.CompilerParams(use_tc_tiling_on_sc=False),
    scratch_types=[
        pltpu.VMEM((CMAX, CHUNK), jnp.int32),
        pltpu.VMEM((CMAX, CHUNK), jnp.int32),
        pltpu.VMEM((2 * NBUF, CHUNK, HID), jnp.float32),
        pltpu.VMEM_SHARED((N_PAD, HID), jnp.float32),
        pltpu.VMEM_SHARED((N_PAD, HID), jnp.float32),
        pltpu.VMEM((1, HID), jnp.float32),
        [pltpu.SemaphoreType.DMA] * (2 * NBUF),
    ],
)
def _msg_pass2(acc1, dinv16, b1m, src2d, dst2d, out, src_v, dst_v, rows_v,
               acc_sh, tab_sh, b1_v, sems):
    cid = lax.axis_index("c")
    sid = lax.axis_index("s")
    chunk0, n_groups = _chunk_layout(cid, sid)
    _load_idx_chunks(cid, chunk0, src2d, dst2d, src_v, dst_v)
    pltpu.sync_copy(b1m, b1_v)

    for k in range(STAGE_ROWS // SCHUNK):
        r0 = sid * STAGE_ROWS + k * SCHUNK
        pltpu.sync_copy(acc1.at[0, pl.ds(r0, SCHUNK)],
                        rows_v.at[0, pl.ds(0, SCHUNK)])
        pltpu.sync_copy(acc1.at[1, pl.ds(r0, SCHUNK)],
                        rows_v.at[1, pl.ds(0, SCHUNK)])
        pltpu.sync_copy(dinv16.at[pl.ds(r0, SCHUNK)],
                        rows_v.at[2, pl.ds(0, SCHUNK)])

        def _row(i, carry):
            d = rows_v[2, i, :]
            pre = (rows_v[0, i, :] + rows_v[1, i, :]) * d + b1_v[0, :]
            rows_v[3, i, :] = jnp.maximum(pre, 0.0) * d
            return carry

        lax.fori_loop(0, SCHUNK, _row, 0)
        pltpu.sync_copy(rows_v.at[3, pl.ds(0, SCHUNK)],
                        tab_sh.at[pl.ds(r0, SCHUNK)])

        @pl.when(cid == 0)
        def _():
            pltpu.sync_copy(rows_v.at[3, pl.ds(0, SCHUNK)],
                            acc_sh.at[pl.ds(r0, SCHUNK)])

    _zero_acc(cid, sid, rows_v, acc_sh)
    plsc.subcore_barrier()
    _msg_loop(cid, sid, out, src_v, dst_v, rows_v, acc_sh, tab_sh, sems,
              n_groups)


@functools.partial(
    pl.kernel,
    mesh=_MESH,
    out_type=jax.ShapeDtypeStruct((N_CORES, N_PAD), jnp.float32),
    compiler_params=pltpu.CompilerParams(use_tc_tiling_on_sc=False),
    scratch_types=[
        pltpu.VMEM((CMAX, CHUNK), jnp.int32),
        pltpu.VMEM((CHUNK,), jnp.float32),
        pltpu.VMEM_SHARED((N_PAD,), jnp.float32),
        [pltpu.SemaphoreType.DMA] * (2 * NBUF),
    ],
)
def _degrees(dst2d, out, dst_v, ones_v, deg_sh, sems):
    cid = lax.axis_index("c")
    sid = lax.axis_index("s")
    chunk0, n_groups = _chunk_layout(cid, sid)

    @pl.when(cid == 0)
    def _():
        pltpu.sync_copy(dst2d.at[pl.ds(chunk0, C0_CHUNKS)],
                        dst_v.at[pl.ds(0, C0_CHUNKS)])

    @pl.when(cid == 1)
    def _():
        pltpu.sync_copy(dst2d.at[pl.ds(chunk0, C1_CHUNKS)],
                        dst_v.at[pl.ds(0, C1_CHUNKS)])

    def _fill(val):
        def _f(i, carry):
            ones_v[pl.ds(i * 16, 16)] = jnp.full((16,), val, jnp.float32)
            return carry
        lax.fori_loop(0, CHUNK // 16, _f, 0)

    _fill(0.0)
    for k in range(ROWS_PER_TILE // CHUNK):
        pltpu.sync_copy(
            ones_v, deg_sh.at[pl.ds(sid * ROWS_PER_TILE + k * CHUNK, CHUNK)])
    _fill(1.0)
    plsc.subcore_barrier()

    def _phase(g, par):
        for b in range(NBUF):
            pltpu.async_copy(ones_v, deg_sh.at[dst_v.at[g * NBUF + b]],
                             sems[par * NBUF + b], add=True)
        for b in range(NBUF):
            @pl.when(g >= 1)
            def _():
                pltpu.make_async_copy(
                    ones_v, deg_sh.at[dst_v.at[(g - 1) * NBUF + b]],
                    sems[(1 - par) * NBUF + b]).wait()

    def _pair(t, carry):
        _phase(t * 2, 0)
        _phase(t * 2 + 1, 1)
        return carry

    lax.fori_loop(0, n_groups // 2, _pair, 0)
    for b in range(NBUF):
        pltpu.make_async_copy(
            ones_v, deg_sh.at[dst_v.at[(n_groups - 1) * NBUF + b]],
            sems[NBUF + b]).wait()
    plsc.subcore_barrier()

    pltpu.sync_copy(
        deg_sh.at[pl.ds(sid * ROWS_PER_TILE, ROWS_PER_TILE)],
        out.at[cid, pl.ds(sid * ROWS_PER_TILE, ROWS_PER_TILE)])


def _tc_a_body(deg_ref, x_ref, w1_ref, g1_ref, dinv_ref):
    deg = deg_ref[:, 0:1] + deg_ref[:, 1:2] + 1.0
    dinv = lax.rsqrt(deg)
    h = jnp.dot(x_ref[...], w1_ref[...], preferred_element_type=jnp.float32)
    g1_ref[...] = h * dinv
    dinv_ref[...] = jnp.broadcast_to(dinv, (BLK, HID))


def _tc_c_body(acc_ref, dinv_ref, w2_ref, b2_ref, out_ref):
    s = (acc_ref[0] + acc_ref[1]) * dinv_ref[...]
    out_ref[...] = (
        jnp.dot(s, w2_ref[...], preferred_element_type=jnp.float32)
        + b2_ref[...])


_tc_a = pl.pallas_call(
    _tc_a_body,
    grid=(GRID,),
    in_specs=[
        pl.BlockSpec((BLK, N_CORES), lambda i: (i, 0)),
        pl.BlockSpec((BLK, IN_DIM), lambda i: (i, 0)),
        pl.BlockSpec((IN_DIM, HID), lambda i: (0, 0)),
    ],
    out_specs=[
        pl.BlockSpec((BLK, HID), lambda i: (i, 0)),
        pl.BlockSpec((BLK, HID), lambda i: (i, 0)),
    ],
    out_shape=[
        jax.ShapeDtypeStruct((N_NODES, HID), jnp.float32),
        jax.ShapeDtypeStruct((N_NODES, HID), jnp.float32),
    ],
)

_tc_c = pl.pallas_call(
    _tc_c_body,
    grid=(GRID,),
    in_specs=[
        pl.BlockSpec((N_CORES, BLK, HID), lambda i: (0, i, 0)),
        pl.BlockSpec((BLK, HID), lambda i: (i, 0)),
        pl.BlockSpec((HID, OUT_DIM), lambda i: (0, 0)),
        pl.BlockSpec((1, OUT_DIM), lambda i: (0, 0)),
    ],
    out_specs=pl.BlockSpec((BLK, OUT_DIM), lambda i: (i, 0)),
    out_shape=jax.ShapeDtypeStruct((N_NODES, OUT_DIM), jnp.float32),
)


def kernel(x, edge_index, W1, b1, W2, b2):
    src = edge_index[0]
    dst = edge_index[1]
    pad_src = jnp.zeros((E_PAD - E_EDGES,), jnp.int32)
    pad_dst = jnp.full((E_PAD - E_EDGES,), N_NODES, jnp.int32)
    src_p = jnp.concatenate([src, pad_src]).reshape(E_PAD // CHUNK, CHUNK)
    dst_p = jnp.concatenate([dst, pad_dst]).reshape(E_PAD // CHUNK, CHUNK)

    deg_parts = _degrees(dst_p)
    g1, dinv16 = _tc_a(deg_parts.T, x, W1)
    acc1 = _msg_pass1(g1, src_p, dst_p)
    acc2 = _msg_pass2(acc1, dinv16, b1[None, :], src_p, dst_p)
    return _tc_c(acc2, dinv16, W2, b2[None, :])


# speedup vs baseline: 1.6886x; 1.0420x over previous
import functools

import jax
import jax.numpy as jnp
from jax import lax
from jax.experimental import pallas as pl
from jax.experimental.pallas import tpu as pltpu
from jax.experimental.pallas import tpu_sc as plsc

N_NODES = 10000
N_PAD = 10240
E_EDGES = 320000
E_PAD = 327680
IN_DIM = 128
HID = 16
OUT_DIM = 3

CHUNK = 128
N_CORES = 2
N_SUB = 16
ROWS_PER_TILE = N_PAD // N_SUB
STAGE_ROWS = N_NODES // N_SUB
SCHUNK = 125

BLK = 1000
GRID = N_NODES // BLK

_MESH = plsc.VectorSubcoreMesh(core_axis_name="c", subcore_axis_name="s")


NBUF = 8

C0_CHUNKS = 96
C1_CHUNKS = 64
CMAX = max(C0_CHUNKS, C1_CHUNKS)
assert N_SUB * (C0_CHUNKS + C1_CHUNKS) == E_PAD // CHUNK
assert C0_CHUNKS % (2 * NBUF) == 0 and C1_CHUNKS % (2 * NBUF) == 0


def _chunk_layout(cid, sid):
    chunk0 = jnp.where(cid == 0, sid * C0_CHUNKS,
                       N_SUB * C0_CHUNKS + sid * C1_CHUNKS)
    n_groups = jnp.where(cid == 0, C0_CHUNKS // NBUF, C1_CHUNKS // NBUF)
    return chunk0, n_groups


def _load_idx_chunks(cid, chunk0, src2d, dst2d, src_v, dst_v):
    @pl.when(cid == 0)
    def _():
        pltpu.sync_copy(src2d.at[pl.ds(chunk0, C0_CHUNKS)],
                        src_v.at[pl.ds(0, C0_CHUNKS)])
        pltpu.sync_copy(dst2d.at[pl.ds(chunk0, C0_CHUNKS)],
                        dst_v.at[pl.ds(0, C0_CHUNKS)])

    @pl.when(cid == 1)
    def _():
        pltpu.sync_copy(src2d.at[pl.ds(chunk0, C1_CHUNKS)],
                        src_v.at[pl.ds(0, C1_CHUNKS)])
        pltpu.sync_copy(dst2d.at[pl.ds(chunk0, C1_CHUNKS)],
                        dst_v.at[pl.ds(0, C1_CHUNKS)])


def _zero_acc(cid, sid, rows_v, acc_sh):
    def _zrow(i, carry):
        rows_v[0, i, :] = jnp.zeros((HID,), jnp.float32)
        return carry

    lax.fori_loop(0, CHUNK, _zrow, 0)

    @pl.when(cid == 1)
    def _():
        for k in range(ROWS_PER_TILE // CHUNK):
            pltpu.sync_copy(
                rows_v.at[0],
                acc_sh.at[pl.ds(sid * ROWS_PER_TILE + k * CHUNK, CHUNK)])

    @pl.when((cid == 0) & (sid == N_SUB - 1))
    def _():
        pltpu.sync_copy(rows_v.at[0], acc_sh.at[pl.ds(N_NODES, CHUNK)])
        pltpu.sync_copy(
            rows_v.at[0, pl.ds(0, N_PAD - N_NODES - CHUNK)],
            acc_sh.at[pl.ds(N_NODES + CHUNK, N_PAD - N_NODES - CHUNK)])


def _msg_loop(cid, sid, out, src_v, dst_v, rows_v, acc_sh, tab_sh, sems,
              n_groups):
    def _gather(c, slot):
        pltpu.make_async_copy(
            tab_sh.at[src_v.at[c]], rows_v.at[slot], sems[slot]).start()

    def _wait_gather(c, slot):
        pltpu.make_async_copy(
            tab_sh.at[src_v.at[c]], rows_v.at[slot], sems[slot]).wait()

    def _scatter(c, slot):
        pltpu.async_copy(
            rows_v.at[slot], acc_sh.at[dst_v.at[c]], sems[slot], add=True)

    def _wait_scatter(c, slot):
        pltpu.make_async_copy(
            rows_v.at[slot], acc_sh.at[dst_v.at[c]], sems[slot]).wait()

    for b in range(NBUF):
        _gather(b, b)

    def _phase(g, par):
        for b in range(NBUF):
            _wait_gather(g * NBUF + b, par * NBUF + b)
            _scatter(g * NBUF + b, par * NBUF + b)
        for b in range(NBUF):
            slot = (1 - par) * NBUF + b

            @pl.when(g >= 1)
            def _():
                _wait_scatter((g - 1) * NBUF + b, slot)

            @pl.when(g + 1 < n_groups)
            def _():
                _gather((g + 1) * NBUF + b, slot)

    def _pair(t, carry):
        _phase(t * 2, 0)
        _phase(t * 2 + 1, 1)
        return carry

    lax.fori_loop(0, n_groups // 2, _pair, 0)
    for b in range(NBUF):
        _wait_scatter((n_groups - 1) * NBUF + b, NBUF + b)
    plsc.subcore_barrier()

    pltpu.sync_copy(
        acc_sh.at[pl.ds(sid * ROWS_PER_TILE, ROWS_PER_TILE)],
        out.at[cid, pl.ds(sid * ROWS_PER_TILE, ROWS_PER_TILE)])


@functools.partial(
    pl.kernel,
    mesh=_MESH,
    out_type=jax.ShapeDtypeStruct((N_CORES, N_PAD, HID), jnp.float32),
    compiler_params=pltpu.CompilerParams(use_tc_tiling_on_sc=False),
    scratch_types=[
        pltpu.VMEM((CMAX, CHUNK), jnp.int32),
        pltpu.VMEM((CMAX, CHUNK), jnp.int32),
        pltpu.VMEM((2 * NBUF, CHUNK, HID), jnp.float32),
        pltpu.VMEM_SHARED((N_PAD, HID), jnp.float32),
        pltpu.VMEM_SHARED((N_PAD, HID), jnp.float32),
        [pltpu.SemaphoreType.DMA] * (2 * NBUF),
    ],
)
def _msg_pass1(table, src2d, dst2d, out, src_v, dst_v, rows_v, acc_sh,
               tab_sh, sems):
    cid = lax.axis_index("c")
    sid = lax.axis_index("s")
    chunk0, n_groups = _chunk_layout(cid, sid)
    _load_idx_chunks(cid, chunk0, src2d, dst2d, src_v, dst_v)

    r0 = sid * STAGE_ROWS
    pltpu.sync_copy(table.at[pl.ds(r0, STAGE_ROWS)],
                    tab_sh.at[pl.ds(r0, STAGE_ROWS)])

    @pl.when(cid == 0)
    def _():
        pltpu.sync_copy(table.at[pl.ds(r0, STAGE_ROWS)],
                        acc_sh.at[pl.ds(r0, STAGE_ROWS)])

    _zero_acc(cid, sid, rows_v, acc_sh)
    plsc.subcore_barrier()
    _msg_loop(cid, sid, out, src_v, dst_v, rows_v, acc_sh, tab_sh, sems,
              n_groups)


@functools.partial(
    pl.kernel,
    mesh=_MESH,
    out_type=jax.ShapeDtypeStruct((N_CORES, N_PAD, HID), jnp.float32),
    compiler_params=pltpu.CompilerParams(use_tc_tiling_on_sc=False),
    scratch_types=[
        pltpu.VMEM((CMAX, CHUNK), jnp.int32),
        pltpu.VMEM((CMAX, CHUNK), jnp.int32),
        pltpu.VMEM((2 * NBUF, CHUNK, HID), jnp.float32),
        pltpu.VMEM_SHARED((N_PAD, HID), jnp.float32),
        pltpu.VMEM_SHARED((N_PAD, HID), jnp.float32),
        pltpu.VMEM((1, HID), jnp.float32),
        [pltpu.SemaphoreType.DMA] * (2 * NBUF),
    ],
)
def _msg_pass2(acc1, dinv16, b1m, src2d, dst2d, out, src_v, dst_v, rows_v,
               acc_sh, tab_sh, b1_v, sems):
    cid = lax.axis_index("c")
    sid = lax.axis_index("s")
    chunk0, n_groups = _chunk_layout(cid, sid)
    _load_idx_chunks(cid, chunk0, src2d, dst2d, src_v, dst_v)
    pltpu.sync_copy(b1m, b1_v)

    for k in range(STAGE_ROWS // SCHUNK):
        r0 = sid * STAGE_ROWS + k * SCHUNK
        pltpu.sync_copy(acc1.at[0, pl.ds(r0, SCHUNK)],
                        rows_v.at[0, pl.ds(0, SCHUNK)])
        pltpu.sync_copy(acc1.at[1, pl.ds(r0, SCHUNK)],
                        rows_v.at[1, pl.ds(0, SCHUNK)])
        pltpu.sync_copy(dinv16.at[pl.ds(r0, SCHUNK)],
                        rows_v.at[2, pl.ds(0, SCHUNK)])

        def _row(i, carry):
            for u in range(5):
                r = i * 5 + u
                d = rows_v[2, r, :]
                pre = (rows_v[0, r, :] + rows_v[1, r, :]) * d + b1_v[0, :]
                rows_v[3, r, :] = jnp.maximum(pre, 0.0) * d
            return carry

        lax.fori_loop(0, SCHUNK // 5, _row, 0)
        pltpu.sync_copy(rows_v.at[3, pl.ds(0, SCHUNK)],
                        tab_sh.at[pl.ds(r0, SCHUNK)])

        @pl.when(cid == 0)
        def _():
            pltpu.sync_copy(rows_v.at[3, pl.ds(0, SCHUNK)],
                            acc_sh.at[pl.ds(r0, SCHUNK)])

    _zero_acc(cid, sid, rows_v, acc_sh)
    plsc.subcore_barrier()
    _msg_loop(cid, sid, out, src_v, dst_v, rows_v, acc_sh, tab_sh, sems,
              n_groups)


@functools.partial(
    pl.kernel,
    mesh=_MESH,
    out_type=jax.ShapeDtypeStruct((N_CORES, N_PAD), jnp.float32),
    compiler_params=pltpu.CompilerParams(use_tc_tiling_on_sc=False),
    scratch_types=[
        pltpu.VMEM((CMAX, CHUNK), jnp.int32),
        pltpu.VMEM((CHUNK,), jnp.float32),
        pltpu.VMEM_SHARED((N_PAD,), jnp.float32),
        [pltpu.SemaphoreType.DMA] * (2 * NBUF),
    ],
)
def _degrees(dst2d, out, dst_v, ones_v, deg_sh, sems):
    cid = lax.axis_index("c")
    sid = lax.axis_index("s")
    chunk0, n_groups = _chunk_layout(cid, sid)

    @pl.when(cid == 0)
    def _():
        pltpu.sync_copy(dst2d.at[pl.ds(chunk0, C0_CHUNKS)],
                        dst_v.at[pl.ds(0, C0_CHUNKS)])

    @pl.when(cid == 1)
    def _():
        pltpu.sync_copy(dst2d.at[pl.ds(chunk0, C1_CHUNKS)],
                        dst_v.at[pl.ds(0, C1_CHUNKS)])

    def _fill(val):
        def _f(i, carry):
            ones_v[pl.ds(i * 16, 16)] = jnp.full((16,), val, jnp.float32)
            return carry
        lax.fori_loop(0, CHUNK // 16, _f, 0)

    _fill(0.0)
    for k in range(ROWS_PER_TILE // CHUNK):
        pltpu.sync_copy(
            ones_v, deg_sh.at[pl.ds(sid * ROWS_PER_TILE + k * CHUNK, CHUNK)])
    _fill(1.0)
    plsc.subcore_barrier()

    def _phase(g, par):
        for b in range(NBUF):
            pltpu.async_copy(ones_v, deg_sh.at[dst_v.at[g * NBUF + b]],
                             sems[par * NBUF + b], add=True)
        for b in range(NBUF):
            @pl.when(g >= 1)
            def _():
                pltpu.make_async_copy(
                    ones_v, deg_sh.at[dst_v.at[(g - 1) * NBUF + b]],
                    sems[(1 - par) * NBUF + b]).wait()

    def _pair(t, carry):
        _phase(t * 2, 0)
        _phase(t * 2 + 1, 1)
        return carry

    lax.fori_loop(0, n_groups // 2, _pair, 0)
    for b in range(NBUF):
        pltpu.make_async_copy(
            ones_v, deg_sh.at[dst_v.at[(n_groups - 1) * NBUF + b]],
            sems[NBUF + b]).wait()
    plsc.subcore_barrier()

    pltpu.sync_copy(
        deg_sh.at[pl.ds(sid * ROWS_PER_TILE, ROWS_PER_TILE)],
        out.at[cid, pl.ds(sid * ROWS_PER_TILE, ROWS_PER_TILE)])


def _tc_a_body(deg_ref, x_ref, w1_ref, g1_ref, dinv_ref):
    deg = deg_ref[:, 0:1] + deg_ref[:, 1:2] + 1.0
    dinv = lax.rsqrt(deg)
    h = jnp.dot(x_ref[...], w1_ref[...], preferred_element_type=jnp.float32)
    g1_ref[...] = h * dinv
    dinv_ref[...] = jnp.broadcast_to(dinv, (BLK, HID))


def _tc_c_body(acc_ref, dinv_ref, w2_ref, b2_ref, out_ref):
    s = (acc_ref[0] + acc_ref[1]) * dinv_ref[...]
    out_ref[...] = (
        jnp.dot(s, w2_ref[...], preferred_element_type=jnp.float32)
        + b2_ref[...])


_tc_a = pl.pallas_call(
    _tc_a_body,
    grid=(GRID,),
    in_specs=[
        pl.BlockSpec((BLK, N_CORES), lambda i: (i, 0)),
        pl.BlockSpec((BLK, IN_DIM), lambda i: (i, 0)),
        pl.BlockSpec((IN_DIM, HID), lambda i: (0, 0)),
    ],
    out_specs=[
        pl.BlockSpec((BLK, HID), lambda i: (i, 0)),
        pl.BlockSpec((BLK, HID), lambda i: (i, 0)),
    ],
    out_shape=[
        jax.ShapeDtypeStruct((N_NODES, HID), jnp.float32),
        jax.ShapeDtypeStruct((N_NODES, HID), jnp.float32),
    ],
)

_tc_c = pl.pallas_call(
    _tc_c_body,
    grid=(GRID,),
    in_specs=[
        pl.BlockSpec((N_CORES, BLK, HID), lambda i: (0, i, 0)),
        pl.BlockSpec((BLK, HID), lambda i: (i, 0)),
        pl.BlockSpec((HID, OUT_DIM), lambda i: (0, 0)),
        pl.BlockSpec((1, OUT_DIM), lambda i: (0, 0)),
    ],
    out_specs=pl.BlockSpec((BLK, OUT_DIM), lambda i: (i, 0)),
    out_shape=jax.ShapeDtypeStruct((N_NODES, OUT_DIM), jnp.float32),
)


def kernel(x, edge_index, W1, b1, W2, b2):
    src = edge_index[0]
    dst = edge_index[1]
    pad_src = jnp.zeros((E_PAD - E_EDGES,), jnp.int32)
    pad_dst = jnp.full((E_PAD - E_EDGES,), N_NODES, jnp.int32)
    src_p = jnp.concatenate([src, pad_src]).reshape(E_PAD // CHUNK, CHUNK)
    dst_p = jnp.concatenate([dst, pad_dst]).reshape(E_PAD // CHUNK, CHUNK)

    deg_parts = _degrees(dst_p)
    g1, dinv16 = _tc_a(deg_parts.T, x, W1)
    acc1 = _msg_pass1(g1, src_p, dst_p)
    acc2 = _msg_pass2(acc1, dinv16, b1[None, :], src_p, dst_p)
    return _tc_c(acc2, dinv16, W2, b2[None, :])


# speedup vs baseline: 1.7930x; 1.0618x over previous
import functools

import jax
import jax.numpy as jnp
from jax import lax
from jax.experimental import pallas as pl
from jax.experimental.pallas import tpu as pltpu
from jax.experimental.pallas import tpu_sc as plsc

N_NODES = 10000
N_PAD = 10240
E_EDGES = 320000
E_PAD = 327680
IN_DIM = 128
HID = 16
OUT_DIM = 3

CHUNK = 128
N_CORES = 2
N_SUB = 16
ROWS_PER_TILE = N_PAD // N_SUB
STAGE_ROWS = N_NODES // N_SUB
SCHUNK = 125

BLK = 1000
GRID = N_NODES // BLK

_MESH = plsc.VectorSubcoreMesh(core_axis_name="c", subcore_axis_name="s")


NBUF = 8

C0_CHUNKS = 96
C1_CHUNKS = 64
CMAX = max(C0_CHUNKS, C1_CHUNKS)
assert N_SUB * (C0_CHUNKS + C1_CHUNKS) == E_PAD // CHUNK
assert C0_CHUNKS % (2 * NBUF) == 0 and C1_CHUNKS % (2 * NBUF) == 0


def _chunk_layout(cid, sid):
    chunk0 = jnp.where(cid == 0, sid * C0_CHUNKS,
                       N_SUB * C0_CHUNKS + sid * C1_CHUNKS)
    n_groups = jnp.where(cid == 0, C0_CHUNKS // NBUF, C1_CHUNKS // NBUF)
    return chunk0, n_groups


def _load_idx_chunks(cid, chunk0, src2d, dst2d, src_v, dst_v):
    @pl.when(cid == 0)
    def _():
        pltpu.sync_copy(src2d.at[pl.ds(chunk0, C0_CHUNKS)],
                        src_v.at[pl.ds(0, C0_CHUNKS)])
        pltpu.sync_copy(dst2d.at[pl.ds(chunk0, C0_CHUNKS)],
                        dst_v.at[pl.ds(0, C0_CHUNKS)])

    @pl.when(cid == 1)
    def _():
        pltpu.sync_copy(src2d.at[pl.ds(chunk0, C1_CHUNKS)],
                        src_v.at[pl.ds(0, C1_CHUNKS)])
        pltpu.sync_copy(dst2d.at[pl.ds(chunk0, C1_CHUNKS)],
                        dst_v.at[pl.ds(0, C1_CHUNKS)])


def _zero_acc(cid, sid, rows_v, acc_sh):
    def _zrow(i, carry):
        rows_v[0, i, :] = jnp.zeros((HID,), jnp.float32)
        return carry

    lax.fori_loop(0, CHUNK, _zrow, 0)

    @pl.when(cid == 1)
    def _():
        for k in range(ROWS_PER_TILE // CHUNK):
            pltpu.sync_copy(
                rows_v.at[0],
                acc_sh.at[pl.ds(sid * ROWS_PER_TILE + k * CHUNK, CHUNK)])

    @pl.when((cid == 0) & (sid == N_SUB - 1))
    def _():
        pltpu.sync_copy(rows_v.at[0], acc_sh.at[pl.ds(N_NODES, CHUNK)])
        pltpu.sync_copy(
            rows_v.at[0, pl.ds(0, N_PAD - N_NODES - CHUNK)],
            acc_sh.at[pl.ds(N_NODES + CHUNK, N_PAD - N_NODES - CHUNK)])


def _msg_loop(cid, sid, out, src_v, dst_v, rows_v, acc_sh, tab_sh, sems,
              n_groups):
    def _gather(c, slot):
        pltpu.make_async_copy(
            tab_sh.at[src_v.at[c]], rows_v.at[slot], sems[slot]).start()

    def _wait_gather(c, slot):
        pltpu.make_async_copy(
            tab_sh.at[src_v.at[c]], rows_v.at[slot], sems[slot]).wait()

    def _scatter(c, slot):
        pltpu.async_copy(
            rows_v.at[slot], acc_sh.at[dst_v.at[c]], sems[slot], add=True)

    def _wait_scatter(c, slot):
        pltpu.make_async_copy(
            rows_v.at[slot], acc_sh.at[dst_v.at[c]], sems[slot]).wait()

    for b in range(NBUF):
        _gather(b, b)

    def _phase(g, par):
        for b in range(NBUF):
            _wait_gather(g * NBUF + b, par * NBUF + b)
            _scatter(g * NBUF + b, par * NBUF + b)
        for b in range(NBUF):
            slot = (1 - par) * NBUF + b

            @pl.when(g >= 1)
            def _():
                _wait_scatter((g - 1) * NBUF + b, slot)

            @pl.when(g + 1 < n_groups)
            def _():
                _gather((g + 1) * NBUF + b, slot)

    def _pair(t, carry):
        _phase(t * 2, 0)
        _phase(t * 2 + 1, 1)
        return carry

    lax.fori_loop(0, n_groups // 2, _pair, 0)
    for b in range(NBUF):
        _wait_scatter((n_groups - 1) * NBUF + b, NBUF + b)
    plsc.subcore_barrier()

    pltpu.sync_copy(
        acc_sh.at[pl.ds(sid * ROWS_PER_TILE, ROWS_PER_TILE)],
        out.at[cid, pl.ds(sid * ROWS_PER_TILE, ROWS_PER_TILE)])


@functools.partial(
    pl.kernel,
    mesh=_MESH,
    out_type=jax.ShapeDtypeStruct((N_CORES, N_PAD, HID), jnp.float32),
    compiler_params=pltpu.CompilerParams(use_tc_tiling_on_sc=False),
    scratch_types=[
        pltpu.VMEM((CMAX, CHUNK), jnp.int32),
        pltpu.VMEM((CMAX, CHUNK), jnp.int32),
        pltpu.VMEM((2 * NBUF, CHUNK, HID), jnp.float32),
        pltpu.VMEM_SHARED((N_PAD, HID), jnp.float32),
        pltpu.VMEM_SHARED((N_PAD, HID), jnp.float32),
        [pltpu.SemaphoreType.DMA] * (2 * NBUF),
    ],
)
def _msg_pass1(table, src2d, dst2d, out, src_v, dst_v, rows_v, acc_sh,
               tab_sh, sems):
    cid = lax.axis_index("c")
    sid = lax.axis_index("s")
    chunk0, n_groups = _chunk_layout(cid, sid)
    _load_idx_chunks(cid, chunk0, src2d, dst2d, src_v, dst_v)

    r0 = sid * STAGE_ROWS
    pltpu.sync_copy(table.at[pl.ds(r0, STAGE_ROWS)],
                    tab_sh.at[pl.ds(r0, STAGE_ROWS)])

    @pl.when(cid == 0)
    def _():
        pltpu.sync_copy(table.at[pl.ds(r0, STAGE_ROWS)],
                        acc_sh.at[pl.ds(r0, STAGE_ROWS)])

    _zero_acc(cid, sid, rows_v, acc_sh)
    plsc.subcore_barrier()
    _msg_loop(cid, sid, out, src_v, dst_v, rows_v, acc_sh, tab_sh, sems,
              n_groups)


@functools.partial(
    pl.kernel,
    mesh=_MESH,
    out_type=jax.ShapeDtypeStruct((N_CORES, N_PAD, HID), jnp.float32),
    compiler_params=pltpu.CompilerParams(use_tc_tiling_on_sc=False),
    scratch_types=[
        pltpu.VMEM((CMAX, CHUNK), jnp.int32),
        pltpu.VMEM((CMAX, CHUNK), jnp.int32),
        pltpu.VMEM((2 * NBUF, CHUNK, HID), jnp.float32),
        pltpu.VMEM_SHARED((N_PAD, HID), jnp.float32),
        pltpu.VMEM_SHARED((N_PAD, HID), jnp.float32),
        pltpu.VMEM((1, HID), jnp.float32),
        pltpu.VMEM((4, STAGE_ROWS, HID), jnp.float32),
        [pltpu.SemaphoreType.DMA] * (2 * NBUF),
    ],
)
def _msg_pass2(acc1, dinv16, b1m, src2d, dst2d, out, src_v, dst_v, rows_v,
               acc_sh, tab_sh, b1_v, st_v, sems):
    cid = lax.axis_index("c")
    sid = lax.axis_index("s")
    chunk0, n_groups = _chunk_layout(cid, sid)
    _load_idx_chunks(cid, chunk0, src2d, dst2d, src_v, dst_v)
    pltpu.sync_copy(b1m, b1_v)

    r0 = sid * STAGE_ROWS
    pltpu.make_async_copy(acc1.at[0, pl.ds(r0, STAGE_ROWS)], st_v.at[0],
                          sems[0]).start()
    pltpu.make_async_copy(acc1.at[1, pl.ds(r0, STAGE_ROWS)], st_v.at[1],
                          sems[1]).start()
    pltpu.make_async_copy(dinv16.at[pl.ds(r0, STAGE_ROWS)], st_v.at[2],
                          sems[2]).start()
    pltpu.make_async_copy(acc1.at[0, pl.ds(r0, STAGE_ROWS)], st_v.at[0],
                          sems[0]).wait()
    pltpu.make_async_copy(acc1.at[1, pl.ds(r0, STAGE_ROWS)], st_v.at[1],
                          sems[1]).wait()
    pltpu.make_async_copy(dinv16.at[pl.ds(r0, STAGE_ROWS)], st_v.at[2],
                          sems[2]).wait()

    def _row(i, carry):
        for u in range(5):
            r = i * 5 + u
            d = st_v[2, r, :]
            pre = (st_v[0, r, :] + st_v[1, r, :]) * d + b1_v[0, :]
            st_v[3, r, :] = jnp.maximum(pre, 0.0) * d
        return carry

    lax.fori_loop(0, STAGE_ROWS // 5, _row, 0)
    pltpu.sync_copy(st_v.at[3], tab_sh.at[pl.ds(r0, STAGE_ROWS)])

    @pl.when(cid == 0)
    def _():
        pltpu.sync_copy(st_v.at[3], acc_sh.at[pl.ds(r0, STAGE_ROWS)])

    _zero_acc(cid, sid, rows_v, acc_sh)
    plsc.subcore_barrier()
    _msg_loop(cid, sid, out, src_v, dst_v, rows_v, acc_sh, tab_sh, sems,
              n_groups)


@functools.partial(
    pl.kernel,
    mesh=_MESH,
    out_type=jax.ShapeDtypeStruct((N_CORES, N_PAD), jnp.float32),
    compiler_params=pltpu.CompilerParams(use_tc_tiling_on_sc=False),
    scratch_types=[
        pltpu.VMEM((CMAX, CHUNK), jnp.int32),
        pltpu.VMEM((CHUNK,), jnp.float32),
        pltpu.VMEM_SHARED((N_PAD,), jnp.float32),
        [pltpu.SemaphoreType.DMA] * (2 * NBUF),
    ],
)
def _degrees(dst2d, out, dst_v, ones_v, deg_sh, sems):
    cid = lax.axis_index("c")
    sid = lax.axis_index("s")
    chunk0, n_groups = _chunk_layout(cid, sid)

    @pl.when(cid == 0)
    def _():
        pltpu.sync_copy(dst2d.at[pl.ds(chunk0, C0_CHUNKS)],
                        dst_v.at[pl.ds(0, C0_CHUNKS)])

    @pl.when(cid == 1)
    def _():
        pltpu.sync_copy(dst2d.at[pl.ds(chunk0, C1_CHUNKS)],
                        dst_v.at[pl.ds(0, C1_CHUNKS)])

    def _fill(val):
        def _f(i, carry):
            ones_v[pl.ds(i * 16, 16)] = jnp.full((16,), val, jnp.float32)
            return carry
        lax.fori_loop(0, CHUNK // 16, _f, 0)

    _fill(0.0)
    for k in range(ROWS_PER_TILE // CHUNK):
        pltpu.sync_copy(
            ones_v, deg_sh.at[pl.ds(sid * ROWS_PER_TILE + k * CHUNK, CHUNK)])
    _fill(1.0)
    plsc.subcore_barrier()

    def _phase(g, par):
        for b in range(NBUF):
            pltpu.async_copy(ones_v, deg_sh.at[dst_v.at[g * NBUF + b]],
                             sems[par * NBUF + b], add=True)
        for b in range(NBUF):
            @pl.when(g >= 1)
            def _():
                pltpu.make_async_copy(
                    ones_v, deg_sh.at[dst_v.at[(g - 1) * NBUF + b]],
                    sems[(1 - par) * NBUF + b]).wait()

    def _pair(t, carry):
        _phase(t * 2, 0)
        _phase(t * 2 + 1, 1)
        return carry

    lax.fori_loop(0, n_groups // 2, _pair, 0)
    for b in range(NBUF):
        pltpu.make_async_copy(
            ones_v, deg_sh.at[dst_v.at[(n_groups - 1) * NBUF + b]],
            sems[NBUF + b]).wait()
    plsc.subcore_barrier()

    pltpu.sync_copy(
        deg_sh.at[pl.ds(sid * ROWS_PER_TILE, ROWS_PER_TILE)],
        out.at[cid, pl.ds(sid * ROWS_PER_TILE, ROWS_PER_TILE)])


def _tc_a_body(deg_ref, x_ref, w1_ref, g1_ref, dinv_ref):
    deg = deg_ref[:, 0:1] + deg_ref[:, 1:2] + 1.0
    dinv = lax.rsqrt(deg)
    h = jnp.dot(x_ref[...], w1_ref[...], preferred_element_type=jnp.float32)
    g1_ref[...] = h * dinv
    dinv_ref[...] = jnp.broadcast_to(dinv, (BLK, HID))


def _tc_c_body(acc_ref, dinv_ref, w2_ref, b2_ref, out_ref):
    s = (acc_ref[0] + acc_ref[1]) * dinv_ref[...]
    out_ref[...] = (
        jnp.dot(s, w2_ref[...], preferred_element_type=jnp.float32)
        + b2_ref[...])


_tc_a = pl.pallas_call(
    _tc_a_body,
    grid=(GRID,),
    in_specs=[
        pl.BlockSpec((BLK, N_CORES), lambda i: (i, 0)),
        pl.BlockSpec((BLK, IN_DIM), lambda i: (i, 0)),
        pl.BlockSpec((IN_DIM, HID), lambda i: (0, 0)),
    ],
    out_specs=[
        pl.BlockSpec((BLK, HID), lambda i: (i, 0)),
        pl.BlockSpec((BLK, HID), lambda i: (i, 0)),
    ],
    out_shape=[
        jax.ShapeDtypeStruct((N_NODES, HID), jnp.float32),
        jax.ShapeDtypeStruct((N_NODES, HID), jnp.float32),
    ],
)

_tc_c = pl.pallas_call(
    _tc_c_body,
    grid=(GRID,),
    in_specs=[
        pl.BlockSpec((N_CORES, BLK, HID), lambda i: (0, i, 0)),
        pl.BlockSpec((BLK, HID), lambda i: (i, 0)),
        pl.BlockSpec((HID, OUT_DIM), lambda i: (0, 0)),
        pl.BlockSpec((1, OUT_DIM), lambda i: (0, 0)),
    ],
    out_specs=pl.BlockSpec((BLK, OUT_DIM), lambda i: (i, 0)),
    out_shape=jax.ShapeDtypeStruct((N_NODES, OUT_DIM), jnp.float32),
)


def kernel(x, edge_index, W1, b1, W2, b2):
    src = edge_index[0]
    dst = edge_index[1]
    pad_src = jnp.zeros((E_PAD - E_EDGES,), jnp.int32)
    pad_dst = jnp.full((E_PAD - E_EDGES,), N_NODES, jnp.int32)
    src_p = jnp.concatenate([src, pad_src]).reshape(E_PAD // CHUNK, CHUNK)
    dst_p = jnp.concatenate([dst, pad_dst]).reshape(E_PAD // CHUNK, CHUNK)

    deg_parts = _degrees(dst_p)
    g1, dinv16 = _tc_a(deg_parts.T, x, W1)
    acc1 = _msg_pass1(g1, src_p, dst_p)
    acc2 = _msg_pass2(acc1, dinv16, b1[None, :], src_p, dst_p)
    return _tc_c(acc2, dinv16, W2, b2[None, :])
